# baseline (speedup 1.0000x reference)
import functools

import jax
import jax.numpy as jnp
from jax import lax
from jax.experimental import pallas as pl
from jax.experimental.pallas import tpu as pltpu
from jax.experimental.pallas import tpu_sc as plsc

N = 10000
E = 160000
D = 128
STEPS = 16
ROWS = 2 * N
ROWS_PAD = 20480
NODES_PER_TILE = 640
NODES_PER_PASS = 320
UNROLL = 4
OFF_LEN = 664
OFF_PAD = 31 * NODES_PER_TILE + OFF_LEN
E2 = 2 * E
K = 128
EPAD = E2 + 2 * K
BLK = 1024
NSUB = 8


def _node_step_body(s_ref, c_ref, w2p_ref, a_ref, b_ref, ca_ref, cb_ref,
                    ya_ref, yb_ref):
    hid = jnp.maximum(
        jnp.dot(s_ref[...], w2p_ref[...], preferred_element_type=jnp.float32)
        + c_ref[...], 0.0)
    ya_ref[...] = jnp.dot(hid, a_ref[...],
                          preferred_element_type=jnp.float32) + ca_ref[...]
    yb_ref[...] = jnp.dot(hid, b_ref[...],
                          preferred_element_type=jnp.float32) + cb_ref[...]


def _node_step(s, cnode, w2p, a, b, ca, cb):
    grid = ROWS_PAD // BLK
    blk_row = pl.BlockSpec((BLK, D), lambda i: (i, 0))
    blk_w = pl.BlockSpec((D, D), lambda i: (0, 0))
    blk_v = pl.BlockSpec((1, D), lambda i: (0, 0))
    return pl.pallas_call(
        _node_step_body,
        grid=(grid,),
        in_specs=[blk_row, blk_row, blk_w, blk_w, blk_w, blk_v, blk_v],
        out_specs=[blk_row, blk_row],
        out_shape=[jax.ShapeDtypeStruct((ROWS_PAD, D), jnp.float32)] * 2,
    )(s, cnode, w2p, a, b, ca.reshape(1, D), cb.reshape(1, D))


def _logit_body(ea_ref, eb_ref, w1_ref, b1_ref, w2_ref, b2_ref, out_ref):
    h = jnp.concatenate([ea_ref[...], eb_ref[...]], axis=1)
    h = jnp.maximum(
        jnp.dot(h, w1_ref[...], preferred_element_type=jnp.float32)
        + b1_ref[...], 0.0)
    out_ref[...] = jnp.dot(h, w2_ref[...],
                           preferred_element_type=jnp.float32) + b2_ref[...]


def _logits(ea, eb, w1, b1, w2, b2):
    bsz = ea.shape[0]
    return pl.pallas_call(
        _logit_body,
        out_shape=jax.ShapeDtypeStruct((bsz, 1), jnp.float32),
    )(ea, eb, w1, b1.reshape(1, D), w2, b2.reshape(1, 1))


def _edge_body(ya_hbm, yb_hbm, src_hbm, dst_hbm, off_hbm, s_hbm,
               yabuf, sbuf, rowbuf, srcbuf, dstbuf, offbuf, sem0, sem1):
    nc_ax = lax.axis_index("c")
    ns_ax = lax.axis_index("s")
    wid = ns_ax * 2 + nc_ax
    base = wid * NODES_PER_TILE

    pltpu.sync_copy(off_hbm.at[pl.ds(base, OFF_LEN)], offbuf)

    def off_at(i):
        return offbuf[pl.ds(i, 16)][0]

    zeros = jnp.zeros((16,), jnp.float32)

    for p in range(NODES_PER_TILE // NODES_PER_PASS):
        basep = base + p * NODES_PER_PASS
        loff = p * NODES_PER_PASS
        pltpu.sync_copy(ya_hbm.at[pl.ds(basep, NODES_PER_PASS), :], yabuf)

        def zero_row(i, _):
            for j in range(NSUB):
                sbuf[i, pl.ds(j * 16, 16)] = zeros
            return 0
        lax.fori_loop(0, NODES_PER_PASS + 1, zero_row, 0)

        e_lo = off_at(jnp.int32(loff))
        e_hi = off_at(jnp.int32(loff + NODES_PER_PASS))
        e0 = (e_lo // 8) * 8
        nchunks = (e_hi - e0 + K - 1) // K

        def issue(c):
            slot = lax.rem(c, 2)
            idx_sl = dstbuf.at[pl.ds(slot * K, K)]
            row_sl = rowbuf.at[pl.ds(slot * K, K), :]
            pltpu.sync_copy(src_hbm.at[pl.ds(e0 + c * K, K)],
                            srcbuf.at[pl.ds(slot * K, K)])
            pltpu.sync_copy(dst_hbm.at[pl.ds(e0 + c * K, K)], idx_sl)

            def start(sem):
                pltpu.make_async_copy(yb_hbm.at[idx_sl], row_sl, sem).start()
            pl.when(slot == 0)(lambda: start(sem0))
            pl.when(slot == 1)(lambda: start(sem1))

        def wait(c):
            slot = lax.rem(c, 2)
            idx_sl = dstbuf.at[pl.ds(slot * K, K)]
            row_sl = rowbuf.at[pl.ds(slot * K, K), :]

            def dowait(sem):
                pltpu.make_async_copy(yb_hbm.at[idx_sl], row_sl, sem).wait()
            pl.when(slot == 0)(lambda: dowait(sem0))
            pl.when(slot == 1)(lambda: dowait(sem1))

        pl.when(nchunks > 0)(lambda: issue(0))

        def chunk_body(c, _):
            wait(c)
            pl.when(c + 1 < nchunks)(lambda: issue(c + 1))
            lo = jnp.maximum(e_lo, e0 + c * K)
            hi = jnp.minimum(e_hi, e0 + (c + 1) * K)
            sbase = lax.rem(c, 2) * K
            ebase = e0 + c * K

            def grp_body(q, _):
                for u in range(UNROLL):
                    ridx = sbase + q * UNROLL + u
                    e = ebase + q * UNROLL + u
                    lvr = srcbuf[pl.ds(ridx, 16)][0] - basep
                    valid = jnp.logical_and(e >= lo, e < hi)
                    lv = jnp.where(valid, lvr, NODES_PER_PASS)
                    for j in range(NSUB):
                        row = rowbuf[ridx, pl.ds(j * 16, 16)]
                        ya = yabuf[jnp.minimum(lv, NODES_PER_PASS - 1),
                                   pl.ds(j * 16, 16)]
                        h = jnp.maximum(ya + row, 0.0)
                        plsc.addupdate(sbuf.at[lv, pl.ds(j * 16, 16)], h)
                return 0
            return lax.fori_loop(0, K // UNROLL, grp_body, 0)

        lax.fori_loop(0, nchunks, chunk_body, 0)
        pltpu.sync_copy(sbuf.at[pl.ds(0, NODES_PER_PASS), :],
                        s_hbm.at[pl.ds(basep, NODES_PER_PASS), :])


def _edge_phase(ya, yb, src, dst, off):
    mesh = plsc.VectorSubcoreMesh(core_axis_name="c", subcore_axis_name="s")
    return pl.kernel(
        _edge_body,
        mesh=mesh,
        out_type=jax.ShapeDtypeStruct((ROWS_PAD, D), jnp.float32),
        scratch_types=[
            pltpu.VMEM((NODES_PER_PASS, D), jnp.float32),
            pltpu.VMEM((NODES_PER_PASS + 8, D), jnp.float32),
            pltpu.VMEM((2 * K, D), jnp.float32),
            pltpu.VMEM((2 * K + 16,), jnp.int32),
            pltpu.VMEM((2 * K,), jnp.int32),
            pltpu.VMEM((OFF_LEN,), jnp.int32),
            pltpu.SemaphoreType.DMA,
            pltpu.SemaphoreType.DMA,
        ],
    )(ya, yb, src, dst, off)


def kernel(nodes_a, edges_a, heads_a, nodes_b, edges_b, heads_b,
           emb, msg_W1, msg_b1, msg_W2, msg_b2,
           post_W1, post_b1, post_W2, post_b2,
           logit_W1, logit_b1, logit_W2, logit_b2):
    wa, wb = msg_W1[:D], msg_W1[D:2 * D]
    pw1a, pw1b = post_W1[:D], post_W1[D:]
    w2p = msg_W2 @ pw1a
    a_w = post_W2 @ wa
    b_w = post_W2 @ wb
    ca = post_b2 @ wa + msg_b1
    cb = post_b2 @ wb

    nodes = jnp.concatenate([nodes_a, nodes_b])
    x0 = jnp.take(emb, nodes, axis=0)

    e_all = jnp.concatenate(
        [edges_a, edges_b.astype(jnp.int32) + N], axis=0).astype(jnp.int32)
    order = jnp.argsort(e_all[:, 0])
    src_s = e_all[order, 0]
    src = jnp.zeros((EPAD,), jnp.int32).at[:E2].set(src_s)
    dst = jnp.zeros((EPAD,), jnp.int32).at[:E2].set(e_all[order, 1])
    off = jnp.full((OFF_PAD,), E2, jnp.int32).at[:ROWS + 1].set(
        jnp.searchsorted(src_s, jnp.arange(ROWS + 1, dtype=jnp.int32)
                         ).astype(jnp.int32))
    deg = (off[1:ROWS + 1] - off[:ROWS]).astype(jnp.float32)

    cnode = deg[:, None] * (msg_b2 @ pw1a)[None, :] + x0 @ pw1b + post_b1
    cnode = jnp.zeros((ROWS_PAD, D), jnp.float32).at[:ROWS].set(cnode)

    ya = jnp.zeros((ROWS_PAD, D), jnp.float32).at[:ROWS].set(x0 @ wa + msg_b1)
    yb = jnp.zeros((ROWS_PAD, D), jnp.float32).at[:ROWS].set(x0 @ wb)

    def step(_, carry):
        ya, yb = carry
        s = _edge_phase(ya, yb, src, dst, off)
        return _node_step(s, cnode, w2p, a_w, b_w, ca, cb)

    ya, yb = lax.fori_loop(0, STEPS - 1, step, (ya, yb))

    s = _edge_phase(ya, yb, src, dst, off)
    x, _ = _node_step(s, cnode, w2p, post_W2, post_W2, post_b2, post_b2)

    ea = jnp.take(x, heads_a, axis=0)
    eb = jnp.take(x, heads_b + N, axis=0)
    return _logits(ea, eb, logit_W1, logit_b1, logit_W2, logit_b2)


# speedup vs baseline: 1.4511x; 1.4511x over previous
import functools

import jax
import jax.numpy as jnp
from jax import lax
from jax.experimental import pallas as pl
from jax.experimental.pallas import tpu as pltpu
from jax.experimental.pallas import tpu_sc as plsc

N = 10000
E = 160000
D = 128
STEPS = 16
ROWS = 2 * N
ROWS_PAD = 20480
NODES_PER_TILE = 640
NODES_PER_PASS = 320
UNROLL = 4
OFF_LEN = 664
OFF_PAD = 31 * NODES_PER_TILE + OFF_LEN
E2 = 2 * E
K = 128
EPAD = E2 + 2 * K
BLK = 1024
NSUB = 8


def _node_step_body(s_ref, c_ref, w2p_ref, a_ref, b_ref, ca_ref, cb_ref,
                    ya_ref, yb_ref):
    hid = jnp.maximum(
        jnp.dot(s_ref[...], w2p_ref[...], preferred_element_type=jnp.float32)
        + c_ref[...], 0.0)
    ya_ref[...] = jnp.dot(hid, a_ref[...],
                          preferred_element_type=jnp.float32) + ca_ref[...]
    yb_ref[...] = jnp.dot(hid, b_ref[...],
                          preferred_element_type=jnp.float32) + cb_ref[...]


def _node_step(s, cnode, w2p, a, b, ca, cb):
    grid = ROWS_PAD // BLK
    blk_row = pl.BlockSpec((BLK, D), lambda i: (i, 0))
    blk_w = pl.BlockSpec((D, D), lambda i: (0, 0))
    blk_v = pl.BlockSpec((1, D), lambda i: (0, 0))
    return pl.pallas_call(
        _node_step_body,
        grid=(grid,),
        in_specs=[blk_row, blk_row, blk_w, blk_w, blk_w, blk_v, blk_v],
        out_specs=[blk_row, blk_row],
        out_shape=[jax.ShapeDtypeStruct((ROWS_PAD, D), jnp.float32)] * 2,
    )(s, cnode, w2p, a, b, ca.reshape(1, D), cb.reshape(1, D))


def _logit_body(ea_ref, eb_ref, w1_ref, b1_ref, w2_ref, b2_ref, out_ref):
    h = jnp.concatenate([ea_ref[...], eb_ref[...]], axis=1)
    h = jnp.maximum(
        jnp.dot(h, w1_ref[...], preferred_element_type=jnp.float32)
        + b1_ref[...], 0.0)
    out_ref[...] = jnp.dot(h, w2_ref[...],
                           preferred_element_type=jnp.float32) + b2_ref[...]


def _logits(ea, eb, w1, b1, w2, b2):
    bsz = ea.shape[0]
    return pl.pallas_call(
        _logit_body,
        out_shape=jax.ShapeDtypeStruct((bsz, 1), jnp.float32),
    )(ea, eb, w1, b1.reshape(1, D), w2, b2.reshape(1, 1))


def _edge_body(ya_hbm, yb_hbm, src_hbm, dst_hbm, off_hbm, s_hbm,
               yabuf, sbuf, rowbuf, srcbuf, dstbuf, offbuf, sem0, sem1):
    nc_ax = lax.axis_index("c")
    ns_ax = lax.axis_index("s")
    wid = ns_ax * 2 + nc_ax
    base = wid * NODES_PER_TILE

    pltpu.sync_copy(off_hbm.at[pl.ds(base, OFF_LEN)], offbuf)

    def off_at(i):
        return offbuf[pl.ds(i, 16)][0]

    zeros = jnp.zeros((16,), jnp.float32)

    for p in range(NODES_PER_TILE // NODES_PER_PASS):
        basep = base + p * NODES_PER_PASS
        loff = p * NODES_PER_PASS
        pltpu.sync_copy(ya_hbm.at[pl.ds(basep, NODES_PER_PASS), :], yabuf)

        def zero_row(i, _):
            for j in range(NSUB):
                sbuf[i, pl.ds(j * 16, 16)] = zeros
            return 0
        lax.fori_loop(0, NODES_PER_PASS + 1, zero_row, 0)

        e_lo = off_at(jnp.int32(loff))
        e_hi = off_at(jnp.int32(loff + NODES_PER_PASS))
        e0 = (e_lo // 8) * 8
        nchunks = (e_hi - e0 + K - 1) // K

        def issue(c):
            slot = lax.rem(c, 2)
            idx_sl = dstbuf.at[pl.ds(slot * K, K)]
            row_sl = rowbuf.at[pl.ds(slot * K, K), :]
            pltpu.sync_copy(src_hbm.at[pl.ds(e0 + c * K, K)],
                            srcbuf.at[pl.ds(slot * K, K)])
            pltpu.sync_copy(dst_hbm.at[pl.ds(e0 + c * K, K)], idx_sl)

            def start(sem):
                pltpu.make_async_copy(yb_hbm.at[idx_sl], row_sl, sem).start()
            pl.when(slot == 0)(lambda: start(sem0))
            pl.when(slot == 1)(lambda: start(sem1))

        def wait(c):
            slot = lax.rem(c, 2)
            idx_sl = dstbuf.at[pl.ds(slot * K, K)]
            row_sl = rowbuf.at[pl.ds(slot * K, K), :]

            def dowait(sem):
                pltpu.make_async_copy(yb_hbm.at[idx_sl], row_sl, sem).wait()
            pl.when(slot == 0)(lambda: dowait(sem0))
            pl.when(slot == 1)(lambda: dowait(sem1))

        pl.when(nchunks > 0)(lambda: issue(0))

        def chunk_body(c, _):
            wait(c)
            pl.when(c + 1 < nchunks)(lambda: issue(c + 1))
            lo = jnp.maximum(e_lo, e0 + c * K)
            hi = jnp.minimum(e_hi, e0 + (c + 1) * K)
            sbase = lax.rem(c, 2) * K
            ebase = e0 + c * K

            @plsc.parallel_loop(0, K, unroll=UNROLL)
            def _edges(i):
                ridx = sbase + i
                e = ebase + i
                lvr = srcbuf[pl.ds(ridx, 16)][0] - basep
                valid = jnp.logical_and(e >= lo, e < hi)
                lv = jnp.where(valid, lvr, NODES_PER_PASS)
                for j in range(NSUB):
                    row = rowbuf[ridx, pl.ds(j * 16, 16)]
                    ya = yabuf[jnp.minimum(lv, NODES_PER_PASS - 1),
                               pl.ds(j * 16, 16)]
                    h = jnp.maximum(ya + row, 0.0)
                    plsc.addupdate(sbuf.at[lv, pl.ds(j * 16, 16)], h)
            return 0

        lax.fori_loop(0, nchunks, chunk_body, 0)
        pltpu.sync_copy(sbuf.at[pl.ds(0, NODES_PER_PASS), :],
                        s_hbm.at[pl.ds(basep, NODES_PER_PASS), :])


def _edge_phase(ya, yb, src, dst, off):
    mesh = plsc.VectorSubcoreMesh(core_axis_name="c", subcore_axis_name="s")
    return pl.kernel(
        _edge_body,
        mesh=mesh,
        out_type=jax.ShapeDtypeStruct((ROWS_PAD, D), jnp.float32),
        scratch_types=[
            pltpu.VMEM((NODES_PER_PASS, D), jnp.float32),
            pltpu.VMEM((NODES_PER_PASS + 8, D), jnp.float32),
            pltpu.VMEM((2 * K, D), jnp.float32),
            pltpu.VMEM((2 * K + 16,), jnp.int32),
            pltpu.VMEM((2 * K,), jnp.int32),
            pltpu.VMEM((OFF_LEN,), jnp.int32),
            pltpu.SemaphoreType.DMA,
            pltpu.SemaphoreType.DMA,
        ],
    )(ya, yb, src, dst, off)


def kernel(nodes_a, edges_a, heads_a, nodes_b, edges_b, heads_b,
           emb, msg_W1, msg_b1, msg_W2, msg_b2,
           post_W1, post_b1, post_W2, post_b2,
           logit_W1, logit_b1, logit_W2, logit_b2):
    wa, wb = msg_W1[:D], msg_W1[D:2 * D]
    pw1a, pw1b = post_W1[:D], post_W1[D:]
    w2p = msg_W2 @ pw1a
    a_w = post_W2 @ wa
    b_w = post_W2 @ wb
    ca = post_b2 @ wa + msg_b1
    cb = post_b2 @ wb

    nodes = jnp.concatenate([nodes_a, nodes_b])
    x0 = jnp.take(emb, nodes, axis=0)

    e_all = jnp.concatenate(
        [edges_a, edges_b.astype(jnp.int32) + N], axis=0).astype(jnp.int32)
    order = jnp.argsort(e_all[:, 0])
    src_s = e_all[order, 0]
    src = jnp.zeros((EPAD,), jnp.int32).at[:E2].set(src_s)
    dst = jnp.zeros((EPAD,), jnp.int32).at[:E2].set(e_all[order, 1])
    off = jnp.full((OFF_PAD,), E2, jnp.int32).at[:ROWS + 1].set(
        jnp.searchsorted(src_s, jnp.arange(ROWS + 1, dtype=jnp.int32)
                         ).astype(jnp.int32))
    deg = (off[1:ROWS + 1] - off[:ROWS]).astype(jnp.float32)

    cnode = deg[:, None] * (msg_b2 @ pw1a)[None, :] + x0 @ pw1b + post_b1
    cnode = jnp.zeros((ROWS_PAD, D), jnp.float32).at[:ROWS].set(cnode)

    ya = jnp.zeros((ROWS_PAD, D), jnp.float32).at[:ROWS].set(x0 @ wa + msg_b1)
    yb = jnp.zeros((ROWS_PAD, D), jnp.float32).at[:ROWS].set(x0 @ wb)

    def step(_, carry):
        ya, yb = carry
        s = _edge_phase(ya, yb, src, dst, off)
        return _node_step(s, cnode, w2p, a_w, b_w, ca, cb)

    ya, yb = lax.fori_loop(0, STEPS - 1, step, (ya, yb))

    s = _edge_phase(ya, yb, src, dst, off)
    x, _ = _node_step(s, cnode, w2p, post_W2, post_W2, post_b2, post_b2)

    ea = jnp.take(x, heads_a, axis=0)
    eb = jnp.take(x, heads_b + N, axis=0)
    return _logits(ea, eb, logit_W1, logit_b1, logit_W2, logit_b2)
